# baseline (speedup 1.0000x reference)
import functools

import jax
import jax.numpy as jnp
from jax import lax
from jax.experimental import pallas as pl
from jax.experimental.pallas import tpu as pltpu
from jax.experimental.pallas import tpu_sc as plsc

N = 50000
E = 800000
DIN = 128
DE = 16
HID = 64
HEADS = 4
C = HID // HEADS
L = 3

NSC = 2
NSUB = 16
LANE = 16
NPASS = 2

ROWW = 24
DROW = 16
CH = 128
NE2 = 802816
EPAD = NE2 - E
EPS = NE2 // NSUB
NCHUNK = EPS // CH
NPAIR = NCHUNK // 2
RPS = 3120
RREM = N - NSUB * RPS

_GD = lax.GatherDimensionNumbers(offset_dims=(), collapsed_slice_dims=(0,),
                                 start_index_map=(0,))


def _vgather(v, idx):
    return lax.gather(v, idx[:, None], _GD, (1,),
                      mode=lax.GatherScatterMode.PROMISE_IN_BOUNDS)


BLK = 2000
BE = 16000


def _write_tabs(h, Wg_ref, ws_ref, wd_ref, tabA_ref, tabD_ref):
    hp = jnp.dot(h, Wg_ref[...], preferred_element_type=jnp.float32)
    als = jnp.dot(h, ws_ref[...], preferred_element_type=jnp.float32)
    ald = jnp.dot(h, wd_ref[...], preferred_element_type=jnp.float32)
    b = h.shape[0]
    z6 = jnp.zeros((b, ROWW - 18), jnp.float32)
    rows = [jnp.concatenate([hp[:, 16 * hd:16 * hd + 16],
                             als[:, hd:hd + 1], als[:, hd:hd + 1], z6],
                            axis=1)
            for hd in range(HEADS)]
    tabA_ref[...] = jnp.stack([jnp.stack(rows[0:2]), jnp.stack(rows[2:4])])
    zd14 = jnp.zeros((b, 14), jnp.float32)
    d0 = jnp.concatenate([zd14, ald[:, 0:2]], axis=1)
    d1 = jnp.concatenate([zd14, ald[:, 2:4]], axis=1)
    tabD_ref[...] = jnp.stack([d0, d1])


def _pre_body(x_ref, Wn_ref, bn_ref, Wg_ref, ws_ref, wd_ref,
              h_ref, tabA_ref, tabD_ref):
    h = jnp.dot(x_ref[...], Wn_ref[...],
                preferred_element_type=jnp.float32) + bn_ref[...]
    h_ref[...] = h
    _write_tabs(h, Wg_ref, ws_ref, wd_ref, tabA_ref, tabD_ref)


def _pre(x, Wn, bn, Wg0, ws0, wd0):
    return pl.pallas_call(
        _pre_body,
        grid=(N // BLK,),
        in_specs=[
            pl.BlockSpec((BLK, DIN), lambda i: (i, 0)),
            pl.BlockSpec((DIN, HID), lambda i: (0, 0)),
            pl.BlockSpec((1, HID), lambda i: (0, 0)),
            pl.BlockSpec((HID, HID), lambda i: (0, 0)),
            pl.BlockSpec((HID, HEADS), lambda i: (0, 0)),
            pl.BlockSpec((HID, HEADS), lambda i: (0, 0)),
        ],
        out_specs=[
            pl.BlockSpec((BLK, HID), lambda i: (i, 0)),
            pl.BlockSpec((NSC, NPASS, BLK, ROWW), lambda i: (0, 0, i, 0)),
            pl.BlockSpec((NSC, BLK, DROW), lambda i: (0, i, 0)),
        ],
        out_shape=[
            jax.ShapeDtypeStruct((N, HID), jnp.float32),
            jax.ShapeDtypeStruct((NSC, NPASS, N, ROWW), jnp.float32),
            jax.ShapeDtypeStruct((NSC, N, DROW), jnp.float32),
        ],
    )(x, Wn, bn.reshape(1, HID), Wg0, ws0, wd0)


def _ale_body(ea_ref, Wc_ref, bc_ref, ale_ref, sum_ref):
    j = pl.program_id(0)
    ea = ea_ref[...]
    prod = jnp.dot(ea, Wc_ref[...], preferred_element_type=jnp.float32)
    ale_ref[...] = prod + bc_ref[...]

    @pl.when(j == 0)
    def _():
        sum_ref[...] = jnp.zeros_like(sum_ref)

    sum_ref[...] += jnp.sum(ea, axis=0, keepdims=True)


def _ale(edge_attr, Wc, bc):
    return pl.pallas_call(
        _ale_body,
        grid=(E // BE,),
        in_specs=[
            pl.BlockSpec((BE, DE), lambda j: (j, 0)),
            pl.BlockSpec((DE, L * HEADS), lambda j: (0, 0)),
            pl.BlockSpec((1, L * HEADS), lambda j: (0, 0)),
        ],
        out_specs=[
            pl.BlockSpec((BE, L * HEADS), lambda j: (j, 0)),
            pl.BlockSpec((1, DE), lambda j: (0, 0)),
        ],
        out_shape=[
            jax.ShapeDtypeStruct((E, L * HEADS), jnp.float32),
            jax.ShapeDtypeStruct((1, DE), jnp.float32),
        ],
    )(edge_attr, Wc, bc.reshape(1, L * HEADS))


def _fuse_body(acc_ref, res_ref, Wg_ref, ws_ref, wd_ref, alel_ref,
               bg_ref, g_ref, b_ref, Wgn_ref, wsn_ref, wdn_ref,
               h_ref, tabA_ref=None, tabD_ref=None, *, has_next):
    res = res_ref[...]
    hp = jnp.dot(res, Wg_ref[...], preferred_element_type=jnp.float32)
    als = jnp.dot(res, ws_ref[...], preferred_element_type=jnp.float32)
    ald = jnp.dot(res, wd_ref[...], preferred_element_type=jnp.float32)
    slog = als + ald + alel_ref[...]
    sa = jnp.exp(jnp.maximum(slog, 0.2 * slog))
    num = jnp.concatenate([acc_ref[0, 0][:, 1:17], acc_ref[0, 1][:, 1:17],
                           acc_ref[1, 0][:, 1:17], acc_ref[1, 1][:, 1:17]],
                          axis=1)
    den = jnp.concatenate([acc_ref[0, 0][:, 0:1], acc_ref[0, 1][:, 0:1],
                           acc_ref[1, 0][:, 0:1], acc_ref[1, 1][:, 0:1]],
                          axis=1)
    num = num + hp * jnp.repeat(sa, C, axis=1)
    den = den + sa
    o = num / (jnp.repeat(den, C, axis=1) + 1e-16) + bg_ref[...] + res
    mu = jnp.mean(o, axis=-1, keepdims=True)
    var = jnp.mean((o - mu) ** 2, axis=-1, keepdims=True)
    o = (o - mu) / jnp.sqrt(var + 1e-5) * g_ref[...] + b_ref[...]
    h = jnp.maximum(o, 0.0)
    h_ref[...] = h
    if has_next:
        _write_tabs(h, Wgn_ref, wsn_ref, wdn_ref, tabA_ref, tabD_ref)


def _fuse(acc, res, Wg_l, ws_l, wd_l, alel, bg_l, g_l, b_l,
          Wg_n, ws_n, wd_n, has_next):
    out_specs = [pl.BlockSpec((BLK, HID), lambda i: (i, 0))]
    out_shape = [jax.ShapeDtypeStruct((N, HID), jnp.float32)]
    if has_next:
        out_specs += [
            pl.BlockSpec((NSC, NPASS, BLK, ROWW), lambda i: (0, 0, i, 0)),
            pl.BlockSpec((NSC, BLK, DROW), lambda i: (0, i, 0)),
        ]
        out_shape += [
            jax.ShapeDtypeStruct((NSC, NPASS, N, ROWW), jnp.float32),
            jax.ShapeDtypeStruct((NSC, N, DROW), jnp.float32),
        ]
    return pl.pallas_call(
        functools.partial(_fuse_body, has_next=has_next),
        grid=(N // BLK,),
        in_specs=[
            pl.BlockSpec((NSC, NPASS, BLK, ROWW), lambda i: (0, 0, i, 0)),
            pl.BlockSpec((BLK, HID), lambda i: (i, 0)),
            pl.BlockSpec((HID, HID), lambda i: (0, 0)),
            pl.BlockSpec((HID, HEADS), lambda i: (0, 0)),
            pl.BlockSpec((HID, HEADS), lambda i: (0, 0)),
            pl.BlockSpec((1, HEADS), lambda i: (0, 0)),
            pl.BlockSpec((1, HID), lambda i: (0, 0)),
            pl.BlockSpec((1, HID), lambda i: (0, 0)),
            pl.BlockSpec((1, HID), lambda i: (0, 0)),
            pl.BlockSpec((HID, HID), lambda i: (0, 0)),
            pl.BlockSpec((HID, HEADS), lambda i: (0, 0)),
            pl.BlockSpec((HID, HEADS), lambda i: (0, 0)),
        ],
        out_specs=out_specs,
        out_shape=out_shape,
    )(acc, res, Wg_l, ws_l, wd_l, alel.reshape(1, HEADS),
      bg_l.reshape(1, HID), g_l.reshape(1, HID), b_l.reshape(1, HID),
      Wg_n, ws_n, wd_n)


def _sc_edge(tabA, tabD, src, dst, ale_l, zeros):
    mesh = plsc.VectorSubcoreMesh(core_axis_name="c", subcore_axis_name="s",
                                  num_cores=NSC, num_subcores=NSUB)

    bank_scratch = [
        pltpu.VMEM((CH,), jnp.int32),
        pltpu.VMEM((CH,), jnp.int32),
        pltpu.VMEM((CH,), jnp.int32),
        pltpu.VMEM((2 * CH + 24,), jnp.float32),
        pltpu.VMEM((CH, ROWW), jnp.float32),
        pltpu.VMEM((CH, DROW), jnp.float32),
        pltpu.VMEM((CH, ROWW), jnp.float32),
        pltpu.SemaphoreType.DMA,
        pltpu.SemaphoreType.DMA,
        pltpu.SemaphoreType.DMA,
    ]

    @functools.partial(
        pl.kernel,
        out_type=jax.ShapeDtypeStruct((NSC * NPASS * N, ROWW), jnp.float32),
        mesh=mesh,
        compiler_params=pltpu.CompilerParams(use_tc_tiling_on_sc=False),
        scratch_types=[pltpu.VMEM_SHARED((N, ROWW), jnp.float32)]
        + bank_scratch + bank_scratch,
    )
    def k(tabA_h, tabD_h, src_h, dst_h, ale_h, z_h, out_h, acc, *banks):
        c = lax.axis_index("c")
        s = lax.axis_index("s")
        r0 = s * RPS

        iota = lax.iota(jnp.int32, LANE)
        mask0 = jnp.where(iota < 1, 1.0, 0.0).astype(jnp.float32)
        ebase = s * EPS
        dcoff = jnp.full((LANE,), c * N, jnp.int32)
        b0 = banks[:10]
        b1 = banks[10:]

        pltpu.sync_copy(z_h.at[pl.ds(0, CH)], b0[6])
        pltpu.sync_copy(z_h.at[pl.ds(0, CH)], b1[6])

        def issue_a(bank, ki):
            srcv, dstv, _, alev, _, _, _, semA, _, _ = bank
            base = ebase + ki * CH
            pltpu.async_copy(src_h.at[pl.ds(base, CH)], srcv, semA)
            pltpu.async_copy(dst_h.at[pl.ds(base, CH)], dstv, semA)
            pltpu.async_copy(ale_h.at[pl.ds(2 * (c * NE2 + base), 2 * CH)],
                             alev.at[pl.ds(16, 2 * CH)], semA)

        def wait_a(bank, ki):
            srcv, dstv, _, alev, _, _, _, semA, _, _ = bank
            base = ebase + ki * CH
            pltpu.make_async_copy(src_h.at[pl.ds(base, CH)], srcv,
                                  semA).wait()
            pltpu.make_async_copy(dst_h.at[pl.ds(base, CH)], dstv,
                                  semA).wait()
            pltpu.make_async_copy(
                ale_h.at[pl.ds(2 * (c * NE2 + base), 2 * CH)],
                alev.at[pl.ds(16, 2 * CH)], semA).wait()

        def fix_issue_b(bank, acoffv):
            srcv, dstv, dgt, _, gA, gD, _, _, semB, _ = bank
            for g in range(CH // LANE):
                sl = pl.ds(g * LANE, LANE)
                srcv[sl] = srcv[sl] + acoffv
                dgt[sl] = dstv[sl] + dcoff
            pltpu.async_copy(tabA_h.at[srcv], gA, semB)
            pltpu.async_copy(tabD_h.at[dgt], gD, semB)

        def wait_b(bank):
            srcv, _, dgt, _, gA, gD, _, _, semB, _ = bank
            pltpu.make_async_copy(tabA_h.at[srcv], gA, semB).wait()
            pltpu.make_async_copy(tabD_h.at[dgt], gD, semB).wait()

        def compute(bank, lanep):
            _, _, _, alev, gA, gD, ob, _, _, _ = bank
            for e in range(CH):
                q = (gA[e, pl.ds(2, LANE)] + gD[e, pl.ds(0, LANE)]
                     + alev[pl.ds(2 * e + 2, LANE)])
                a = jnp.exp(jnp.maximum(q, 0.2 * q))
                b = _vgather(a, lanep)
                ob[e, pl.ds(0, LANE)] = b * mask0
                ob[e, pl.ds(1, LANE)] = gA[e, pl.ds(0, LANE)] * b

        def issue_s(bank):
            _, dstv, _, _, _, _, ob, _, _, semS = bank
            pltpu.async_copy(ob, acc.at[dstv], semS, add=True)

        def wait_s(bank):
            _, dstv, _, _, _, _, ob, _, _, semS = bank
            pltpu.make_async_copy(ob, acc.at[dstv], semS).wait()

        def one_pass(p, carry0):
            pltpu.sync_copy(z_h.at[pl.ds(r0, RPS)], acc.at[pl.ds(r0, RPS)])

            @pl.when(s == 0)
            def _():
                pltpu.sync_copy(z_h.at[pl.ds(NSUB * RPS, RREM)],
                                acc.at[pl.ds(NSUB * RPS, RREM)])

            plsc.subcore_barrier()

            lanep = jnp.full((LANE,), 14, jnp.int32) + p
            acoff = (2 * c + p) * N
            acoffv = jnp.full((LANE,), 0, jnp.int32) + acoff

            issue_a(b0, 0)
            wait_a(b0, 0)
            fix_issue_b(b0, acoffv)

            def pair(i, carry):
                k0 = 2 * i
                issue_a(b1, k0 + 1)

                @pl.when(i > 0)
                def _():
                    wait_s(b0)

                wait_b(b0)
                compute(b0, lanep)
                issue_s(b0)
                wait_a(b1, k0 + 1)
                fix_issue_b(b1, acoffv)

                @pl.when(k0 + 2 < NCHUNK)
                def _():
                    issue_a(b0, k0 + 2)

                @pl.when(i > 0)
                def _():
                    wait_s(b1)

                wait_b(b1)
                compute(b1, lanep)
                issue_s(b1)

                @pl.when(k0 + 2 < NCHUNK)
                def _():
                    wait_a(b0, k0 + 2)
                    fix_issue_b(b0, acoffv)

                return carry

            lax.fori_loop(0, NPAIR, pair, 0)
            wait_s(b0)
            wait_s(b1)
            plsc.subcore_barrier()
            pltpu.sync_copy(acc.at[pl.ds(r0, RPS)],
                            out_h.at[pl.ds(acoff + r0, RPS)])

            @pl.when(s == 0)
            def _():
                pltpu.sync_copy(acc.at[pl.ds(NSUB * RPS, RREM)],
                                out_h.at[pl.ds(acoff + NSUB * RPS, RREM)])

            plsc.subcore_barrier()
            return carry0

        lax.fori_loop(0, NPASS, one_pass, 0)

    return k(tabA, tabD, src, dst, ale_l, zeros)


def kernel(x, edge_index, edge_attr, Wn, bn, We, be, Wg, a_src, a_dst,
           Weg, a_eg, bg, ln_g, ln_b):
    src = edge_index[0]
    dst = edge_index[1]

    we_all = jnp.einsum('ldhc,lhc->ldh', Weg.reshape(L, HID, HEADS, C), a_eg)
    ws_all = jnp.einsum('ldhc,lhc->ldh', Wg.reshape(L, HID, HEADS, C), a_src)
    wd_all = jnp.einsum('ldhc,lhc->ldh', Wg.reshape(L, HID, HEADS, C), a_dst)
    W_all2 = jnp.moveaxis(we_all, 0, 1).reshape(HID, L * HEADS)
    Wc = We @ W_all2
    bc = be @ W_all2

    ale_12, easum = _ale(edge_attr, Wc, bc)
    ea_mean = (easum[0] / E) @ We + be
    ale_loop = (ea_mean @ W_all2).reshape(L, HEADS)

    ale_t = jnp.transpose(ale_12.reshape(E, L, NSC, 2), (1, 2, 0, 3))
    ale_pad = jnp.full((L, NSC, EPAD, 2), -1e30, jnp.float32)
    ale_p = jnp.concatenate([ale_t, ale_pad], axis=2)

    srcp = jnp.concatenate([src, jnp.zeros((EPAD,), src.dtype)])
    dstp = jnp.concatenate([dst, jnp.zeros((EPAD,), dst.dtype)])

    zeros = jnp.zeros((N, ROWW), jnp.float32)

    h, tabA, tabD = _pre(x, Wn, bn, Wg[0], ws_all[0], wd_all[0])
    for l in range(L):
        has_next = l < L - 1
        acc = _sc_edge(tabA.reshape(NSC * NPASS * N, ROWW),
                       tabD.reshape(NSC * N, DROW),
                       srcp, dstp, ale_p[l].reshape(-1), zeros)
        acc = acc.reshape(NSC, NPASS, N, ROWW)
        nxt = min(l + 1, L - 1)
        outs = _fuse(acc, h, Wg[l], ws_all[l], wd_all[l], ale_loop[l],
                     bg[l], ln_g[l], ln_b[l],
                     Wg[nxt], ws_all[nxt], wd_all[nxt], has_next)
        if has_next:
            h, tabA, tabD = outs
        else:
            h = outs[0]
    return h


# speedup vs baseline: 1.0879x; 1.0879x over previous
import functools

import jax
import jax.numpy as jnp
from jax import lax
from jax.experimental import pallas as pl
from jax.experimental.pallas import tpu as pltpu
from jax.experimental.pallas import tpu_sc as plsc

N = 50000
E = 800000
DIN = 128
DE = 16
HID = 64
HEADS = 4
C = HID // HEADS
L = 3

NSC = 2
NSUB = 16
LANE = 16
NPASS = 2

ROWW = 24
DROW = 16
CH = 128
NE2 = 802816
EPAD = NE2 - E
EPS = NE2 // NSUB
NCHUNK = EPS // CH
NPAIR = NCHUNK // 2
RPS = 3120
RREM = N - NSUB * RPS

_GD = lax.GatherDimensionNumbers(offset_dims=(), collapsed_slice_dims=(0,),
                                 start_index_map=(0,))


def _vgather(v, idx):
    return lax.gather(v, idx[:, None], _GD, (1,),
                      mode=lax.GatherScatterMode.PROMISE_IN_BOUNDS)


BLK = 2000
BE = 16000


def _write_tabs(h, Ma_ref, Md_ref, tabA_ref, tabD_ref):
    f32 = jnp.float32
    rows = [[jnp.dot(h, Ma_ref[cc, pp], preferred_element_type=f32)
             for pp in range(NPASS)] for cc in range(NSC)]
    tabA_ref[...] = jnp.stack([jnp.stack(rows[0]), jnp.stack(rows[1])])
    tabD_ref[...] = jnp.stack(
        [jnp.dot(h, Md_ref[cc], preferred_element_type=f32)
         for cc in range(NSC)])


def _pre_body(x_ref, Wn_ref, bn_ref, Ma_ref, Md_ref,
              h_ref, tabA_ref, tabD_ref):
    h = jnp.dot(x_ref[...], Wn_ref[...],
                preferred_element_type=jnp.float32) + bn_ref[...]
    h_ref[...] = h
    _write_tabs(h, Ma_ref, Md_ref, tabA_ref, tabD_ref)


def _pre(x, Wn, bn, Ma0, Md0):
    return pl.pallas_call(
        _pre_body,
        grid=(N // BLK,),
        in_specs=[
            pl.BlockSpec((BLK, DIN), lambda i: (i, 0)),
            pl.BlockSpec((DIN, HID), lambda i: (0, 0)),
            pl.BlockSpec((1, HID), lambda i: (0, 0)),
            pl.BlockSpec((NSC, NPASS, HID, ROWW), lambda i: (0, 0, 0, 0)),
            pl.BlockSpec((NSC, HID, DROW), lambda i: (0, 0, 0)),
        ],
        out_specs=[
            pl.BlockSpec((BLK, HID), lambda i: (i, 0)),
            pl.BlockSpec((NSC, NPASS, BLK, ROWW), lambda i: (0, 0, i, 0)),
            pl.BlockSpec((NSC, BLK, DROW), lambda i: (0, i, 0)),
        ],
        out_shape=[
            jax.ShapeDtypeStruct((N, HID), jnp.float32),
            jax.ShapeDtypeStruct((NSC, NPASS, N, ROWW), jnp.float32),
            jax.ShapeDtypeStruct((NSC, N, DROW), jnp.float32),
        ],
    )(x, Wn, bn.reshape(1, HID), Ma0, Md0)


def _ale_body(ea_ref, Wc_ref, bc_ref, ale_ref, sum_ref):
    j = pl.program_id(0)
    ea = ea_ref[...]
    prod = jnp.dot(ea, Wc_ref[...], preferred_element_type=jnp.float32)
    ale_ref[...] = prod + bc_ref[...]

    @pl.when(j == 0)
    def _():
        sum_ref[...] = jnp.zeros_like(sum_ref)

    sum_ref[...] += jnp.sum(ea, axis=0, keepdims=True)


def _ale(edge_attr, Wc, bc):
    return pl.pallas_call(
        _ale_body,
        grid=(E // BE,),
        in_specs=[
            pl.BlockSpec((BE, DE), lambda j: (j, 0)),
            pl.BlockSpec((DE, L * HEADS), lambda j: (0, 0)),
            pl.BlockSpec((1, L * HEADS), lambda j: (0, 0)),
        ],
        out_specs=[
            pl.BlockSpec((BE, L * HEADS), lambda j: (j, 0)),
            pl.BlockSpec((1, DE), lambda j: (0, 0)),
        ],
        out_shape=[
            jax.ShapeDtypeStruct((E, L * HEADS), jnp.float32),
            jax.ShapeDtypeStruct((1, DE), jnp.float32),
        ],
    )(edge_attr, Wc, bc.reshape(1, L * HEADS))


def _fuse_body(acc_ref, res_ref, Wg_ref, ws_ref, wd_ref, alel_ref,
               bg_ref, g_ref, b_ref, P_ref, Q_ref, R_ref,
               Man_ref, Mdn_ref,
               h_ref, tabA_ref=None, tabD_ref=None, *, has_next):
    f32 = jnp.float32
    res = res_ref[...]
    hp = jnp.dot(res, Wg_ref[...], preferred_element_type=f32)
    als = jnp.dot(res, ws_ref[...], preferred_element_type=f32)
    ald = jnp.dot(res, wd_ref[...], preferred_element_type=f32)
    slog = als + ald + alel_ref[...]
    sa = jnp.exp(jnp.maximum(slog, 0.2 * slog))
    num = jnp.zeros((res.shape[0], HID), f32)
    den = jnp.zeros((res.shape[0], HEADS), f32)
    for cc in range(NSC):
        for pp in range(NPASS):
            a = acc_ref[cc, pp]
            num = num + jnp.dot(a, P_ref[cc, pp], preferred_element_type=f32)
            den = den + jnp.dot(a, Q_ref[cc, pp], preferred_element_type=f32)
    num = num + hp * jnp.dot(sa, R_ref[...], preferred_element_type=f32)
    den = den + sa
    denr = jnp.dot(den, R_ref[...], preferred_element_type=f32)
    o = num / (denr + 1e-16) + bg_ref[...] + res
    mu = jnp.mean(o, axis=-1, keepdims=True)
    var = jnp.mean((o - mu) ** 2, axis=-1, keepdims=True)
    o = (o - mu) / jnp.sqrt(var + 1e-5) * g_ref[...] + b_ref[...]
    h = jnp.maximum(o, 0.0)
    h_ref[...] = h
    if has_next:
        _write_tabs(h, Man_ref, Mdn_ref, tabA_ref, tabD_ref)


def _fuse(acc, res, Wg_l, ws_l, wd_l, alel, bg_l, g_l, b_l,
          P, Q, R, Ma_n, Md_n, has_next):
    out_specs = [pl.BlockSpec((BLK, HID), lambda i: (i, 0))]
    out_shape = [jax.ShapeDtypeStruct((N, HID), jnp.float32)]
    if has_next:
        out_specs += [
            pl.BlockSpec((NSC, NPASS, BLK, ROWW), lambda i: (0, 0, i, 0)),
            pl.BlockSpec((NSC, BLK, DROW), lambda i: (0, i, 0)),
        ]
        out_shape += [
            jax.ShapeDtypeStruct((NSC, NPASS, N, ROWW), jnp.float32),
            jax.ShapeDtypeStruct((NSC, N, DROW), jnp.float32),
        ]
    return pl.pallas_call(
        functools.partial(_fuse_body, has_next=has_next),
        grid=(N // BLK,),
        in_specs=[
            pl.BlockSpec((NSC, NPASS, BLK, ROWW), lambda i: (0, 0, i, 0)),
            pl.BlockSpec((BLK, HID), lambda i: (i, 0)),
            pl.BlockSpec((HID, HID), lambda i: (0, 0)),
            pl.BlockSpec((HID, HEADS), lambda i: (0, 0)),
            pl.BlockSpec((HID, HEADS), lambda i: (0, 0)),
            pl.BlockSpec((1, HEADS), lambda i: (0, 0)),
            pl.BlockSpec((1, HID), lambda i: (0, 0)),
            pl.BlockSpec((1, HID), lambda i: (0, 0)),
            pl.BlockSpec((1, HID), lambda i: (0, 0)),
            pl.BlockSpec((NSC, NPASS, ROWW, HID), lambda i: (0, 0, 0, 0)),
            pl.BlockSpec((NSC, NPASS, ROWW, HEADS), lambda i: (0, 0, 0, 0)),
            pl.BlockSpec((HEADS, HID), lambda i: (0, 0)),
            pl.BlockSpec((NSC, NPASS, HID, ROWW), lambda i: (0, 0, 0, 0)),
            pl.BlockSpec((NSC, HID, DROW), lambda i: (0, 0, 0)),
        ],
        out_specs=out_specs,
        out_shape=out_shape,
    )(acc, res, Wg_l, ws_l, wd_l, alel.reshape(1, HEADS),
      bg_l.reshape(1, HID), g_l.reshape(1, HID), b_l.reshape(1, HID),
      P, Q, R, Ma_n, Md_n)


def _sc_edge(tabA, tabD, src, dst, ale_l, zeros):
    mesh = plsc.VectorSubcoreMesh(core_axis_name="c", subcore_axis_name="s",
                                  num_cores=NSC, num_subcores=NSUB)

    bank_scratch = [
        pltpu.VMEM((CH,), jnp.int32),
        pltpu.VMEM((CH,), jnp.int32),
        pltpu.VMEM((CH,), jnp.int32),
        pltpu.VMEM((2 * CH + 24,), jnp.float32),
        pltpu.VMEM((CH, ROWW), jnp.float32),
        pltpu.VMEM((CH, DROW), jnp.float32),
        pltpu.VMEM((CH, ROWW), jnp.float32),
        pltpu.SemaphoreType.DMA,
        pltpu.SemaphoreType.DMA,
        pltpu.SemaphoreType.DMA,
    ]

    @functools.partial(
        pl.kernel,
        out_type=jax.ShapeDtypeStruct((NSC * NPASS * N, ROWW), jnp.float32),
        mesh=mesh,
        compiler_params=pltpu.CompilerParams(use_tc_tiling_on_sc=False),
        scratch_types=[pltpu.VMEM_SHARED((N, ROWW), jnp.float32)]
        + bank_scratch + bank_scratch,
    )
    def k(tabA_h, tabD_h, src_h, dst_h, ale_h, z_h, out_h, acc, *banks):
        c = lax.axis_index("c")
        s = lax.axis_index("s")
        r0 = s * RPS

        iota = lax.iota(jnp.int32, LANE)
        mask0 = jnp.where(iota < 1, 1.0, 0.0).astype(jnp.float32)
        ebase = s * EPS
        dcoff = jnp.full((LANE,), c * N, jnp.int32)
        b0 = banks[:10]
        b1 = banks[10:]

        pltpu.sync_copy(z_h.at[pl.ds(0, CH)], b0[6])
        pltpu.sync_copy(z_h.at[pl.ds(0, CH)], b1[6])

        def issue_a(bank, ki):
            srcv, dstv, _, alev, _, _, _, semA, _, _ = bank
            base = ebase + ki * CH
            pltpu.async_copy(src_h.at[pl.ds(base, CH)], srcv, semA)
            pltpu.async_copy(dst_h.at[pl.ds(base, CH)], dstv, semA)
            pltpu.async_copy(ale_h.at[pl.ds(2 * (c * NE2 + base), 2 * CH)],
                             alev.at[pl.ds(16, 2 * CH)], semA)

        def wait_a(bank, ki):
            srcv, dstv, _, alev, _, _, _, semA, _, _ = bank
            base = ebase + ki * CH
            pltpu.make_async_copy(src_h.at[pl.ds(base, CH)], srcv,
                                  semA).wait()
            pltpu.make_async_copy(dst_h.at[pl.ds(base, CH)], dstv,
                                  semA).wait()
            pltpu.make_async_copy(
                ale_h.at[pl.ds(2 * (c * NE2 + base), 2 * CH)],
                alev.at[pl.ds(16, 2 * CH)], semA).wait()

        def fix_issue_b(bank, acoffv):
            srcv, dstv, dgt, _, gA, gD, _, _, semB, _ = bank
            for g in range(CH // LANE):
                sl = pl.ds(g * LANE, LANE)
                srcv[sl] = srcv[sl] + acoffv
                dgt[sl] = dstv[sl] + dcoff
            pltpu.async_copy(tabA_h.at[srcv], gA, semB)
            pltpu.async_copy(tabD_h.at[dgt], gD, semB)

        def wait_b(bank):
            srcv, _, dgt, _, gA, gD, _, _, semB, _ = bank
            pltpu.make_async_copy(tabA_h.at[srcv], gA, semB).wait()
            pltpu.make_async_copy(tabD_h.at[dgt], gD, semB).wait()

        def compute(bank, lanep):
            _, _, _, alev, gA, gD, ob, _, _, _ = bank
            for e in range(CH):
                q = (gA[e, pl.ds(2, LANE)] + gD[e, pl.ds(0, LANE)]
                     + alev[pl.ds(2 * e + 2, LANE)])
                a = jnp.exp(jnp.maximum(q, 0.2 * q))
                b = _vgather(a, lanep)
                ob[e, pl.ds(0, LANE)] = b * mask0
                ob[e, pl.ds(1, LANE)] = gA[e, pl.ds(0, LANE)] * b

        def issue_s(bank):
            _, dstv, _, _, _, _, ob, _, _, semS = bank
            pltpu.async_copy(ob, acc.at[dstv], semS, add=True)

        def wait_s(bank):
            _, dstv, _, _, _, _, ob, _, _, semS = bank
            pltpu.make_async_copy(ob, acc.at[dstv], semS).wait()

        def one_pass(p, carry0):
            pltpu.sync_copy(z_h.at[pl.ds(r0, RPS)], acc.at[pl.ds(r0, RPS)])

            @pl.when(s == 0)
            def _():
                pltpu.sync_copy(z_h.at[pl.ds(NSUB * RPS, RREM)],
                                acc.at[pl.ds(NSUB * RPS, RREM)])

            plsc.subcore_barrier()

            lanep = jnp.full((LANE,), 14, jnp.int32) + p
            acoff = (2 * c + p) * N
            acoffv = jnp.full((LANE,), 0, jnp.int32) + acoff

            issue_a(b0, 0)
            wait_a(b0, 0)
            fix_issue_b(b0, acoffv)

            def pair(i, carry):
                k0 = 2 * i
                issue_a(b1, k0 + 1)

                @pl.when(i > 0)
                def _():
                    wait_s(b0)

                wait_b(b0)
                compute(b0, lanep)
                issue_s(b0)
                wait_a(b1, k0 + 1)
                fix_issue_b(b1, acoffv)

                @pl.when(k0 + 2 < NCHUNK)
                def _():
                    issue_a(b0, k0 + 2)

                @pl.when(i > 0)
                def _():
                    wait_s(b1)

                wait_b(b1)
                compute(b1, lanep)
                issue_s(b1)

                @pl.when(k0 + 2 < NCHUNK)
                def _():
                    wait_a(b0, k0 + 2)
                    fix_issue_b(b0, acoffv)

                return carry

            lax.fori_loop(0, NPAIR, pair, 0)
            wait_s(b0)
            wait_s(b1)
            plsc.subcore_barrier()
            pltpu.sync_copy(acc.at[pl.ds(r0, RPS)],
                            out_h.at[pl.ds(acoff + r0, RPS)])

            @pl.when(s == 0)
            def _():
                pltpu.sync_copy(acc.at[pl.ds(NSUB * RPS, RREM)],
                                out_h.at[pl.ds(acoff + NSUB * RPS, RREM)])

            plsc.subcore_barrier()
            return carry0

        lax.fori_loop(0, NPASS, one_pass, 0)

    return k(tabA, tabD, src, dst, ale_l, zeros)


def kernel(x, edge_index, edge_attr, Wn, bn, We, be, Wg, a_src, a_dst,
           Weg, a_eg, bg, ln_g, ln_b):
    src = edge_index[0]
    dst = edge_index[1]

    we_all = jnp.einsum('ldhc,lhc->ldh', Weg.reshape(L, HID, HEADS, C), a_eg)
    ws_all = jnp.einsum('ldhc,lhc->ldh', Wg.reshape(L, HID, HEADS, C), a_src)
    wd_all = jnp.einsum('ldhc,lhc->ldh', Wg.reshape(L, HID, HEADS, C), a_dst)
    W_all2 = jnp.moveaxis(we_all, 0, 1).reshape(HID, L * HEADS)
    Wc = We @ W_all2
    bc = be @ W_all2

    ale_12, easum = _ale(edge_attr, Wc, bc)
    ea_mean = (easum[0] / E) @ We + be
    ale_loop = (ea_mean @ W_all2).reshape(L, HEADS)

    ale_t = jnp.transpose(ale_12.reshape(E, L, NSC, 2), (1, 2, 0, 3))
    ale_pad = jnp.full((L, NSC, EPAD, 2), -1e30, jnp.float32)
    ale_p = jnp.concatenate([ale_t, ale_pad], axis=2)

    srcp = jnp.concatenate([src, jnp.zeros((EPAD,), src.dtype)])
    dstp = jnp.concatenate([dst, jnp.zeros((EPAD,), dst.dtype)])

    zeros = jnp.zeros((N, ROWW), jnp.float32)

    def build_ma_md(Wgl, wsl, wdl):
        zc = jnp.zeros((HID, ROWW - 18), jnp.float32)
        ma = jnp.stack([
            jnp.stack([jnp.concatenate(
                [Wgl[:, 16 * (2 * cc + pp):16 * (2 * cc + pp) + 16],
                 wsl[:, 2 * cc + pp:2 * cc + pp + 1],
                 wsl[:, 2 * cc + pp:2 * cc + pp + 1], zc], axis=1)
                for pp in range(NPASS)]) for cc in range(NSC)])
        zd = jnp.zeros((HID, 14), jnp.float32)
        md = jnp.stack([jnp.concatenate([zd, wdl[:, 2 * cc:2 * cc + 2]],
                                        axis=1) for cc in range(NSC)])
        return ma, md

    mamd = [build_ma_md(Wg[l2], ws_all[l2], wd_all[l2]) for l2 in range(L)]

    ey = jnp.eye(16, dtype=jnp.float32)
    P = jnp.zeros((NSC, NPASS, ROWW, HID), jnp.float32)
    Q = jnp.zeros((NSC, NPASS, ROWW, HEADS), jnp.float32)
    for cc in range(NSC):
        for pp in range(NPASS):
            hd = 2 * cc + pp
            P = P.at[cc, pp, 1:17, 16 * hd:16 * hd + 16].set(ey)
            Q = Q.at[cc, pp, 0, hd].set(1.0)
    R = jnp.repeat(jnp.eye(HEADS, dtype=jnp.float32), C, axis=1)

    h, tabA, tabD = _pre(x, Wn, bn, mamd[0][0], mamd[0][1])
    for l in range(L):
        has_next = l < L - 1
        acc = _sc_edge(tabA.reshape(NSC * NPASS * N, ROWW),
                       tabD.reshape(NSC * N, DROW),
                       srcp, dstp, ale_p[l].reshape(-1), zeros)
        acc = acc.reshape(NSC, NPASS, N, ROWW)
        nxt = min(l + 1, L - 1)
        outs = _fuse(acc, h, Wg[l], ws_all[l], wd_all[l], ale_loop[l],
                     bg[l], ln_g[l], ln_b[l],
                     P, Q, R, mamd[nxt][0], mamd[nxt][1], has_next)
        if has_next:
            h, tabA, tabD = outs
        else:
            h = outs[0]
    return h
